# Initial kernel scaffold; baseline (speedup 1.0000x reference)
import jax
import jax.numpy as jnp
from jax.experimental import pallas as pl


def kernel(x, edge_index, W1, b1, W2, b2, Wl, bl):
    raise NotImplementedError("write your pallas kernel here")


# speedup vs baseline: 20.7318x; 20.7318x over previous
import functools

import jax
import jax.numpy as jnp
from jax import lax
from jax.experimental import pallas as pl
from jax.experimental.pallas import tpu as pltpu
from jax.experimental.pallas import tpu_sc as plsc

N = 10000
E = 320000
D = 128

NC = 2
NS = 16
NW = NC * NS

RPT = 632
RPT_LAST = N - (NS - 1) * RPT

EPT_DEG = E // NC // NS
CD = 200
RD = EPT_DEG // CD

EPW = E // NW
CA = 200
RA = EPW // CA

_mesh = plsc.VectorSubcoreMesh(core_axis_name="c", subcore_axis_name="s")


def _tile_sliced_copy(sid, make_src, make_dst):
    @pl.when(sid < NS - 1)
    def _():
        off = sid * RPT
        pltpu.sync_copy(make_src(off, RPT), make_dst(off, RPT))

    @pl.when(sid == NS - 1)
    def _():
        off = (NS - 1) * RPT
        pltpu.sync_copy(make_src(off, RPT_LAST), make_dst(off, RPT_LAST))


EPT = E // NW


@functools.partial(
    pl.kernel,
    out_type=jax.ShapeDtypeStruct((NC, NS, 1, N), jnp.float32),
    mesh=_mesh,
    compiler_params=pltpu.CompilerParams(needs_layout_passes=False),
    scratch_types=[
        pltpu.VMEM((EPT,), jnp.int32),
        pltpu.VMEM((1, N), jnp.float32),
    ],
)
def _deg_kernel(dst_hbm, out_hbm, idx_v, acc_v):
    cid = lax.axis_index("c")
    sid = lax.axis_index("s")

    zero16 = jnp.zeros((16,), jnp.float32)
    zero16i = jnp.zeros((16,), jnp.int32)
    one16 = jnp.full((16,), 1.0, jnp.float32)

    def zfill(i, _):
        acc_v[0, pl.ds(i * 16, 16)] = zero16
        return 0

    lax.fori_loop(0, N // 16, zfill, 0)

    wid = cid * NS + sid
    pltpu.sync_copy(dst_hbm.at[pl.ds(wid * EPT, EPT)], idx_v)

    def body(j, _):
        idx = idx_v[pl.ds(j * 16, 16)]
        plsc.addupdate_scatter(acc_v, [zero16i, idx], one16)
        return 0

    lax.fori_loop(0, EPT // 16, body, 0)

    pltpu.sync_copy(acc_v, out_hbm.at[cid, sid])


@functools.partial(
    pl.kernel,
    out_type=jax.ShapeDtypeStruct((NC, N, D), jnp.float32),
    mesh=_mesh,
    scratch_types=[
        pltpu.VMEM((CA,), jnp.int32),
        pltpu.VMEM((CA,), jnp.int32),
        pltpu.VMEM((CA, D), jnp.float32),
        pltpu.VMEM_SHARED((N, D), jnp.float32),
    ],
)
def _agg_kernel(y_hbm, src_hbm, dst_hbm, out_hbm, idx_s, idx_d, rows_v, z_sh):
    cid = lax.axis_index("c")
    sid = lax.axis_index("s")

    _tile_sliced_copy(
        sid,
        lambda off, n: y_hbm.at[pl.ds(off, n)],
        lambda off, n: z_sh.at[pl.ds(off, n)],
    )
    plsc.subcore_barrier()

    base0 = (cid * NS + sid) * EPW

    def round_body(r, _):
        base = base0 + r * CA
        pltpu.sync_copy(src_hbm.at[pl.ds(base, CA)], idx_s)
        pltpu.sync_copy(dst_hbm.at[pl.ds(base, CA)], idx_d)
        pltpu.sync_copy(y_hbm.at[idx_s], rows_v)
        pltpu.sync_copy(rows_v, z_sh.at[idx_d], add=True)
        return 0

    lax.fori_loop(0, RA, round_body, 0)
    plsc.subcore_barrier()

    _tile_sliced_copy(
        sid,
        lambda off, n: z_sh.at[pl.ds(off, n)],
        lambda off, n: out_hbm.at[cid, pl.ds(off, n)],
    )


_BR = 1000
_GRID = N // _BR


def _dinv_body(degp_ref, dinv_ref):
    deg = jnp.sum(degp_ref[...], axis=(0, 1, 2)) + 1.0
    dinv_ref[...] = lax.rsqrt(deg)[:, None]


def _lin1_body(x_ref, w_ref, dinv_ref, y_ref):
    y = jnp.dot(x_ref[...], w_ref[...], preferred_element_type=jnp.float32)
    y_ref[...] = y * dinv_ref[...]


def _lin2_body(z_ref, y_ref, dinv_ref, b_ref, w_ref, y2_ref):
    dinv = dinv_ref[...]
    z = z_ref[0] + z_ref[1] - y_ref[...]
    h = jnp.tanh(z * dinv + b_ref[...])
    y = jnp.dot(h, w_ref[...], preferred_element_type=jnp.float32)
    y2_ref[...] = y * dinv


def _head_body(z_ref, y_ref, dinv_ref, b_ref, wlt_ref, bl_ref, out_ref, lsm_ref):
    z = z_ref[0] + z_ref[1] - y_ref[...]
    h = jnp.tanh(z * dinv_ref[...] + b_ref[...])
    o = jnp.dot(h, wlt_ref[...], preferred_element_type=jnp.float32) + bl_ref[...]
    m = jnp.max(o, axis=1, keepdims=True)
    lse = jnp.log(jnp.sum(jnp.exp(o - m), axis=1, keepdims=True)) + m
    out_ref[...] = o
    lsm_ref[...] = o - lse


def _row_spec(width):
    return pl.BlockSpec((_BR, width), lambda i: (i, 0))


def _full_spec(shape):
    nz = len(shape)
    return pl.BlockSpec(shape, lambda i, _n=nz: (0,) * _n)


def _z_spec():
    return pl.BlockSpec((NC, _BR, D), lambda i: (0, i, 0))


def kernel(x, edge_index, W1, b1, W2, b2, Wl, bl):
    src = edge_index[0]
    dst = edge_index[1]
    b1r = b1.reshape(1, D)
    b2r = b2.reshape(1, D)
    blr = bl.reshape(1, D)
    wlt = Wl.T

    degp = _deg_kernel(dst)

    dinv = pl.pallas_call(
        _dinv_body,
        out_shape=jax.ShapeDtypeStruct((N, 1), jnp.float32),
    )(degp)

    y1 = pl.pallas_call(
        _lin1_body,
        grid=(_GRID,),
        in_specs=[_row_spec(D), _full_spec((D, D)), _row_spec(1)],
        out_specs=_row_spec(D),
        out_shape=jax.ShapeDtypeStruct((N, D), jnp.float32),
    )(x, W1, dinv)

    z1 = _agg_kernel(y1, src, dst)

    y2 = pl.pallas_call(
        _lin2_body,
        grid=(_GRID,),
        in_specs=[
            _z_spec(),
            _row_spec(D),
            _row_spec(1),
            _full_spec((1, D)),
            _full_spec((D, D)),
        ],
        out_specs=_row_spec(D),
        out_shape=jax.ShapeDtypeStruct((N, D), jnp.float32),
    )(z1, y1, dinv, b1r, W2)

    z2 = _agg_kernel(y2, src, dst)

    out, lsm = pl.pallas_call(
        _head_body,
        grid=(_GRID,),
        in_specs=[
            _z_spec(),
            _row_spec(D),
            _row_spec(1),
            _full_spec((1, D)),
            _full_spec((D, D)),
            _full_spec((1, D)),
        ],
        out_specs=[_row_spec(D), _row_spec(D)],
        out_shape=[
            jax.ShapeDtypeStruct((N, D), jnp.float32),
            jax.ShapeDtypeStruct((N, D), jnp.float32),
        ],
    )(z2, y2, dinv, b2r, wlt, blr)

    return (out, lsm)


# speedup vs baseline: 27.5962x; 1.3311x over previous
import functools

import jax
import jax.numpy as jnp
from jax import lax
from jax.experimental import pallas as pl
from jax.experimental.pallas import tpu as pltpu
from jax.experimental.pallas import tpu_sc as plsc

N = 10000
E = 320000
D = 128

NC = 2
NS = 16
NW = NC * NS

RPT = 632
RPT_LAST = N - (NS - 1) * RPT

EPT_DEG = E // NC // NS
CD = 200
RD = EPT_DEG // CD

EPW = E // NW
CA = 40
RA = EPW // CA
KJ = RA // 4

_mesh = plsc.VectorSubcoreMesh(core_axis_name="c", subcore_axis_name="s")


def _tile_sliced_copy(sid, make_src, make_dst):
    @pl.when(sid < NS - 1)
    def _():
        off = sid * RPT
        pltpu.sync_copy(make_src(off, RPT), make_dst(off, RPT))

    @pl.when(sid == NS - 1)
    def _():
        off = (NS - 1) * RPT
        pltpu.sync_copy(make_src(off, RPT_LAST), make_dst(off, RPT_LAST))


EPT = E // NW


@functools.partial(
    pl.kernel,
    out_type=jax.ShapeDtypeStruct((NC, NS, 1, N), jnp.float32),
    mesh=_mesh,
    compiler_params=pltpu.CompilerParams(needs_layout_passes=False),
    scratch_types=[
        pltpu.VMEM((EPT,), jnp.int32),
        pltpu.VMEM((1, N), jnp.float32),
    ],
)
def _deg_kernel(dst_hbm, out_hbm, idx_v, acc_v):
    cid = lax.axis_index("c")
    sid = lax.axis_index("s")

    zero16 = jnp.zeros((16,), jnp.float32)
    zero16i = jnp.zeros((16,), jnp.int32)
    one16 = jnp.full((16,), 1.0, jnp.float32)

    def zfill(i, _):
        acc_v[0, pl.ds(i * 16, 16)] = zero16
        return 0

    lax.fori_loop(0, N // 16, zfill, 0)

    wid = cid * NS + sid
    pltpu.sync_copy(dst_hbm.at[pl.ds(wid * EPT, EPT)], idx_v)

    def body(j, _):
        idx = idx_v[pl.ds(j * 16, 16)]
        plsc.addupdate_scatter(acc_v, [zero16i, idx], one16)
        return 0

    lax.fori_loop(0, EPT // 16, body, 0)

    pltpu.sync_copy(acc_v, out_hbm.at[cid, sid])


@functools.partial(
    pl.kernel,
    out_type=jax.ShapeDtypeStruct((NC, N, D), jnp.float32),
    mesh=_mesh,
    scratch_types=[
        pltpu.VMEM((EPW,), jnp.int32),
        pltpu.VMEM((EPW,), jnp.int32),
        pltpu.VMEM((CA, D), jnp.float32),
        pltpu.VMEM((CA, D), jnp.float32),
        pltpu.VMEM((CA, D), jnp.float32),
        pltpu.VMEM((CA, D), jnp.float32),
        pltpu.SemaphoreType.DMA,
        pltpu.SemaphoreType.DMA,
        pltpu.SemaphoreType.DMA,
        pltpu.SemaphoreType.DMA,
        pltpu.SemaphoreType.DMA,
        pltpu.SemaphoreType.DMA,
        pltpu.SemaphoreType.DMA,
        pltpu.SemaphoreType.DMA,
        pltpu.VMEM_SHARED((N, D), jnp.float32),
    ],
)
def _agg_kernel(y_hbm, src_hbm, dst_hbm, out_hbm, idx_s, idx_d,
                rows_a, rows_b, rows_c, rows_d,
                gsa, gsb, gsc, gsd, ssa, ssb, ssc, ssd, z_sh):
    cid = lax.axis_index("c")
    sid = lax.axis_index("s")

    wid = cid * NS + sid
    pltpu.sync_copy(src_hbm.at[wid], idx_s)
    pltpu.sync_copy(dst_hbm.at[wid], idx_d)

    _tile_sliced_copy(
        sid,
        lambda off, n: y_hbm.at[pl.ds(off, n)],
        lambda off, n: z_sh.at[pl.ds(off, n)],
    )
    plsc.subcore_barrier()

    rows = [rows_a, rows_b, rows_c, rows_d]
    gsem = [gsa, gsb, gsc, gsd]
    ssem = [ssa, ssb, ssc, ssd]

    def g_start(r, q):
        pltpu.async_copy(
            y_hbm.at[idx_s.at[pl.ds(r * CA, CA)]], rows[q], gsem[q]
        )

    def s_start(r, q):
        pltpu.async_copy(
            rows[q], z_sh.at[idx_d.at[pl.ds(r * CA, CA)]], ssem[q], add=True
        )

    def g_wait(r, q):
        pltpu.make_async_copy(
            y_hbm.at[idx_s.at[pl.ds(r * CA, CA)]], rows[q], gsem[q]
        ).wait()

    def s_wait(r, q):
        pltpu.make_async_copy(
            rows[q], z_sh.at[idx_d.at[pl.ds(r * CA, CA)]], ssem[q]
        ).wait()

    g_start(0, 0)
    g_start(1, 1)

    def quad_body(j, _):
        r0 = 4 * j

        g_wait(r0, 0)
        s_start(r0, 0)

        @pl.when(j > 0)
        def _():
            s_wait(r0 - 2, 2)

        g_start(r0 + 2, 2)

        g_wait(r0 + 1, 1)
        s_start(r0 + 1, 1)

        @pl.when(j > 0)
        def _():
            s_wait(r0 - 1, 3)

        g_start(r0 + 3, 3)

        g_wait(r0 + 2, 2)
        s_start(r0 + 2, 2)
        s_wait(r0, 0)

        @pl.when(r0 + 4 < RA)
        def _():
            g_start(r0 + 4, 0)

        g_wait(r0 + 3, 3)
        s_start(r0 + 3, 3)
        s_wait(r0 + 1, 1)

        @pl.when(r0 + 5 < RA)
        def _():
            g_start(r0 + 5, 1)

        return 0

    lax.fori_loop(0, KJ, quad_body, 0)

    g_wait(RA - 2, 0)
    s_start(RA - 2, 0)
    g_wait(RA - 1, 1)
    s_start(RA - 1, 1)
    s_wait(RA - 4, 2)
    s_wait(RA - 3, 3)
    s_wait(RA - 2, 0)
    s_wait(RA - 1, 1)
    plsc.subcore_barrier()

    _tile_sliced_copy(
        sid,
        lambda off, n: z_sh.at[pl.ds(off, n)],
        lambda off, n: out_hbm.at[cid, pl.ds(off, n)],
    )


_BR = 1000
_GRID = N // _BR


def _dinv_body(degp_ref, dinv_ref):
    deg = jnp.sum(degp_ref[...], axis=(0, 1, 2)) + 1.0
    dinv_ref[...] = lax.rsqrt(deg)[:, None]


def _lin1_body(x_ref, w_ref, dinv_ref, y_ref):
    y = jnp.dot(x_ref[...], w_ref[...], preferred_element_type=jnp.float32)
    y_ref[...] = y * dinv_ref[...]


def _lin2_body(z_ref, y_ref, dinv_ref, b_ref, w_ref, y2_ref):
    dinv = dinv_ref[...]
    z = z_ref[0] + z_ref[1] - y_ref[...]
    h = jnp.tanh(z * dinv + b_ref[...])
    y = jnp.dot(h, w_ref[...], preferred_element_type=jnp.float32)
    y2_ref[...] = y * dinv


def _head_body(z_ref, y_ref, dinv_ref, b_ref, wlt_ref, bl_ref, out_ref, lsm_ref):
    z = z_ref[0] + z_ref[1] - y_ref[...]
    h = jnp.tanh(z * dinv_ref[...] + b_ref[...])
    o = jnp.dot(h, wlt_ref[...], preferred_element_type=jnp.float32) + bl_ref[...]
    m = jnp.max(o, axis=1, keepdims=True)
    lse = jnp.log(jnp.sum(jnp.exp(o - m), axis=1, keepdims=True)) + m
    out_ref[...] = o
    lsm_ref[...] = o - lse


def _row_spec(width):
    return pl.BlockSpec((_BR, width), lambda i: (i, 0))


def _full_spec(shape):
    nz = len(shape)
    return pl.BlockSpec(shape, lambda i, _n=nz: (0,) * _n)


def _z_spec():
    return pl.BlockSpec((NC, _BR, D), lambda i: (0, i, 0))


def kernel(x, edge_index, W1, b1, W2, b2, Wl, bl):
    src = edge_index[0]
    dst = edge_index[1]
    src2 = src.reshape(NW, EPW)
    dst2 = dst.reshape(NW, EPW)
    b1r = b1.reshape(1, D)
    b2r = b2.reshape(1, D)
    blr = bl.reshape(1, D)
    wlt = Wl.T

    degp = _deg_kernel(dst)

    dinv = pl.pallas_call(
        _dinv_body,
        out_shape=jax.ShapeDtypeStruct((N, 1), jnp.float32),
    )(degp)

    y1 = pl.pallas_call(
        _lin1_body,
        grid=(_GRID,),
        in_specs=[_row_spec(D), _full_spec((D, D)), _row_spec(1)],
        out_specs=_row_spec(D),
        out_shape=jax.ShapeDtypeStruct((N, D), jnp.float32),
    )(x, W1, dinv)

    z1 = _agg_kernel(y1, src2, dst2)

    y2 = pl.pallas_call(
        _lin2_body,
        grid=(_GRID,),
        in_specs=[
            _z_spec(),
            _row_spec(D),
            _row_spec(1),
            _full_spec((1, D)),
            _full_spec((D, D)),
        ],
        out_specs=_row_spec(D),
        out_shape=jax.ShapeDtypeStruct((N, D), jnp.float32),
    )(z1, y1, dinv, b1r, W2)

    z2 = _agg_kernel(y2, src2, dst2)

    out, lsm = pl.pallas_call(
        _head_body,
        grid=(_GRID,),
        in_specs=[
            _z_spec(),
            _row_spec(D),
            _row_spec(1),
            _full_spec((1, D)),
            _full_spec((D, D)),
            _full_spec((1, D)),
        ],
        out_specs=[_row_spec(D), _row_spec(D)],
        out_shape=[
            jax.ShapeDtypeStruct((N, D), jnp.float32),
            jax.ShapeDtypeStruct((N, D), jnp.float32),
        ],
    )(z2, y2, dinv, b2r, wlt, blr)

    return (out, lsm)


# speedup vs baseline: 35.8432x; 1.2988x over previous
import functools

import jax
import jax.numpy as jnp
from jax import lax
from jax.experimental import pallas as pl
from jax.experimental.pallas import tpu as pltpu
from jax.experimental.pallas import tpu_sc as plsc

N = 10000
E = 320000
D = 128

NC = 2
NS = 16
NW = NC * NS

RPT = 632
RPT_LAST = N - (NS - 1) * RPT

EPT_DEG = E // NC // NS
CD = 200
RD = EPT_DEG // CD

EPW = E // NW
CA = 40
RA = EPW // CA
NB = 6
LK = 4
KJ = (RA - LK) // NB

_mesh = plsc.VectorSubcoreMesh(core_axis_name="c", subcore_axis_name="s")


def _tile_sliced_copy(sid, make_src, make_dst):
    @pl.when(sid < NS - 1)
    def _():
        off = sid * RPT
        pltpu.sync_copy(make_src(off, RPT), make_dst(off, RPT))

    @pl.when(sid == NS - 1)
    def _():
        off = (NS - 1) * RPT
        pltpu.sync_copy(make_src(off, RPT_LAST), make_dst(off, RPT_LAST))


EPT = E // NW


@functools.partial(
    pl.kernel,
    out_type=jax.ShapeDtypeStruct((NC, NS, 1, N), jnp.float32),
    mesh=_mesh,
    compiler_params=pltpu.CompilerParams(needs_layout_passes=False),
    scratch_types=[
        pltpu.VMEM((EPT,), jnp.int32),
        pltpu.VMEM((1, N), jnp.float32),
    ],
)
def _deg_kernel(dst_hbm, out_hbm, idx_v, acc_v):
    cid = lax.axis_index("c")
    sid = lax.axis_index("s")

    zero16 = jnp.zeros((16,), jnp.float32)
    zero16i = jnp.zeros((16,), jnp.int32)
    one16 = jnp.full((16,), 1.0, jnp.float32)

    def zfill(i, _):
        acc_v[0, pl.ds(i * 16, 16)] = zero16
        return 0

    lax.fori_loop(0, N // 16, zfill, 0)

    wid = cid * NS + sid
    pltpu.sync_copy(dst_hbm.at[pl.ds(wid * EPT, EPT)], idx_v)

    def body(j, _):
        idx = idx_v[pl.ds(j * 16, 16)]
        plsc.addupdate_scatter(acc_v, [zero16i, idx], one16)
        return 0

    lax.fori_loop(0, EPT // 16, body, 0)

    pltpu.sync_copy(acc_v, out_hbm.at[cid, sid])


@functools.partial(
    pl.kernel,
    out_type=jax.ShapeDtypeStruct((NC, N, D), jnp.float32),
    mesh=_mesh,
    scratch_types=[
        pltpu.VMEM((EPW,), jnp.int32),
        pltpu.VMEM((EPW,), jnp.int32),
        pltpu.VMEM((CA, D), jnp.float32),
        pltpu.VMEM((CA, D), jnp.float32),
        pltpu.VMEM((CA, D), jnp.float32),
        pltpu.VMEM((CA, D), jnp.float32),
        pltpu.VMEM((CA, D), jnp.float32),
        pltpu.VMEM((CA, D), jnp.float32),
        pltpu.SemaphoreType.DMA,
        pltpu.SemaphoreType.DMA,
        pltpu.SemaphoreType.DMA,
        pltpu.SemaphoreType.DMA,
        pltpu.SemaphoreType.DMA,
        pltpu.SemaphoreType.DMA,
        pltpu.SemaphoreType.DMA,
        pltpu.SemaphoreType.DMA,
        pltpu.SemaphoreType.DMA,
        pltpu.SemaphoreType.DMA,
        pltpu.SemaphoreType.DMA,
        pltpu.SemaphoreType.DMA,
        pltpu.VMEM_SHARED((N, D), jnp.float32),
    ],
)
def _agg_kernel(y_hbm, src_hbm, dst_hbm, out_hbm, idx_s, idx_d,
                rows_a, rows_b, rows_c, rows_d, rows_e, rows_f,
                gsa, gsb, gsc, gsd, gse, gsf,
                ssa, ssb, ssc, ssd, sse, ssf, z_sh):
    cid = lax.axis_index("c")
    sid = lax.axis_index("s")

    wid = cid * NS + sid
    pltpu.sync_copy(src_hbm.at[wid], idx_s)
    pltpu.sync_copy(dst_hbm.at[wid], idx_d)

    _tile_sliced_copy(
        sid,
        lambda off, n: y_hbm.at[pl.ds(off, n)],
        lambda off, n: z_sh.at[pl.ds(off, n)],
    )
    plsc.subcore_barrier()

    rows = [rows_a, rows_b, rows_c, rows_d, rows_e, rows_f]
    gsem = [gsa, gsb, gsc, gsd, gse, gsf]
    ssem = [ssa, ssb, ssc, ssd, sse, ssf]

    def g_start(r, q):
        pltpu.async_copy(
            y_hbm.at[idx_s.at[pl.ds(r * CA, CA)]], rows[q], gsem[q]
        )

    def s_start(r, q):
        pltpu.async_copy(
            rows[q], z_sh.at[idx_d.at[pl.ds(r * CA, CA)]], ssem[q], add=True
        )

    def g_wait(r, q):
        pltpu.make_async_copy(
            y_hbm.at[idx_s.at[pl.ds(r * CA, CA)]], rows[q], gsem[q]
        ).wait()

    def s_wait(r, q):
        pltpu.make_async_copy(
            rows[q], z_sh.at[idx_d.at[pl.ds(r * CA, CA)]], ssem[q]
        ).wait()

    for q in range(LK):
        g_start(q, q)

    def hex_body(j, _):
        r0 = NB * j
        for i in range(NB):
            ri = r0 + i
            qn = (i + LK) % NB
            g_wait(ri, i)
            s_start(ri, i)
            if i + LK >= NB:
                s_wait(ri + LK - NB, qn)
                g_start(ri + LK, qn)
            else:

                @pl.when(j > 0)
                def _(ri=ri, qn=qn):
                    s_wait(ri + LK - NB, qn)

                g_start(ri + LK, qn)
        return 0

    lax.fori_loop(0, KJ, hex_body, 0)

    for t in range(LK):
        g_wait(RA - LK + t, t)
        s_start(RA - LK + t, t)
    s_wait(RA - LK - 2, NB - 2)
    s_wait(RA - LK - 1, NB - 1)
    for t in range(LK):
        s_wait(RA - LK + t, t)
    plsc.subcore_barrier()

    _tile_sliced_copy(
        sid,
        lambda off, n: z_sh.at[pl.ds(off, n)],
        lambda off, n: out_hbm.at[cid, pl.ds(off, n)],
    )


_BR = 1000
_GRID = N // _BR


def _dinv_body(degp_ref, dinv_ref):
    deg = jnp.sum(degp_ref[...], axis=(0, 1, 2)) + 1.0
    dinv_ref[...] = lax.rsqrt(deg)[:, None]


def _lin1_body(x_ref, w_ref, dinv_ref, y_ref):
    y = jnp.dot(x_ref[...], w_ref[...], preferred_element_type=jnp.float32)
    y_ref[...] = y * dinv_ref[...]


def _lin2_body(z_ref, y_ref, dinv_ref, b_ref, w_ref, y2_ref):
    dinv = dinv_ref[...]
    z = z_ref[0] + z_ref[1] - y_ref[...]
    h = jnp.tanh(z * dinv + b_ref[...])
    y = jnp.dot(h, w_ref[...], preferred_element_type=jnp.float32)
    y2_ref[...] = y * dinv


def _head_body(z_ref, y_ref, dinv_ref, b_ref, wlt_ref, bl_ref, out_ref, lsm_ref):
    z = z_ref[0] + z_ref[1] - y_ref[...]
    h = jnp.tanh(z * dinv_ref[...] + b_ref[...])
    o = jnp.dot(h, wlt_ref[...], preferred_element_type=jnp.float32) + bl_ref[...]
    m = jnp.max(o, axis=1, keepdims=True)
    lse = jnp.log(jnp.sum(jnp.exp(o - m), axis=1, keepdims=True)) + m
    out_ref[...] = o
    lsm_ref[...] = o - lse


def _row_spec(width):
    return pl.BlockSpec((_BR, width), lambda i: (i, 0))


def _full_spec(shape):
    nz = len(shape)
    return pl.BlockSpec(shape, lambda i, _n=nz: (0,) * _n)


def _z_spec():
    return pl.BlockSpec((NC, _BR, D), lambda i: (0, i, 0))


def kernel(x, edge_index, W1, b1, W2, b2, Wl, bl):
    src = edge_index[0]
    dst = edge_index[1]
    src2 = src.reshape(NW, EPW)
    dst2 = dst.reshape(NW, EPW)
    b1r = b1.reshape(1, D)
    b2r = b2.reshape(1, D)
    blr = bl.reshape(1, D)
    wlt = Wl.T

    degp = _deg_kernel(dst)

    dinv = pl.pallas_call(
        _dinv_body,
        out_shape=jax.ShapeDtypeStruct((N, 1), jnp.float32),
    )(degp)

    y1 = pl.pallas_call(
        _lin1_body,
        grid=(_GRID,),
        in_specs=[_row_spec(D), _full_spec((D, D)), _row_spec(1)],
        out_specs=_row_spec(D),
        out_shape=jax.ShapeDtypeStruct((N, D), jnp.float32),
    )(x, W1, dinv)

    z1 = _agg_kernel(y1, src2, dst2)

    y2 = pl.pallas_call(
        _lin2_body,
        grid=(_GRID,),
        in_specs=[
            _z_spec(),
            _row_spec(D),
            _row_spec(1),
            _full_spec((1, D)),
            _full_spec((D, D)),
        ],
        out_specs=_row_spec(D),
        out_shape=jax.ShapeDtypeStruct((N, D), jnp.float32),
    )(z1, y1, dinv, b1r, W2)

    z2 = _agg_kernel(y2, src2, dst2)

    out, lsm = pl.pallas_call(
        _head_body,
        grid=(_GRID,),
        in_specs=[
            _z_spec(),
            _row_spec(D),
            _row_spec(1),
            _full_spec((1, D)),
            _full_spec((D, D)),
            _full_spec((1, D)),
        ],
        out_specs=[_row_spec(D), _row_spec(D)],
        out_shape=[
            jax.ShapeDtypeStruct((N, D), jnp.float32),
            jax.ShapeDtypeStruct((N, D), jnp.float32),
        ],
    )(z2, y2, dinv, b2r, wlt, blr)

    return (out, lsm)


# speedup vs baseline: 36.9175x; 1.0300x over previous
import functools

import jax
import jax.numpy as jnp
from jax import lax
from jax.experimental import pallas as pl
from jax.experimental.pallas import tpu as pltpu
from jax.experimental.pallas import tpu_sc as plsc

N = 10000
E = 320000
D = 128

NC = 2
NS = 16
NW = NC * NS

RPT = 632
RPT_LAST = N - (NS - 1) * RPT

EPT_DEG = E // NC // NS
CD = 200
RD = EPT_DEG // CD

EPW = E // NW
CA = 16
RA = EPW // CA
NB = 11
LK = 9
KJ = (RA - LK) // NB

_mesh = plsc.VectorSubcoreMesh(core_axis_name="c", subcore_axis_name="s")


def _tile_sliced_copy(sid, make_src, make_dst):
    @pl.when(sid < NS - 1)
    def _():
        off = sid * RPT
        pltpu.sync_copy(make_src(off, RPT), make_dst(off, RPT))

    @pl.when(sid == NS - 1)
    def _():
        off = (NS - 1) * RPT
        pltpu.sync_copy(make_src(off, RPT_LAST), make_dst(off, RPT_LAST))


EPT = E // NW


@functools.partial(
    pl.kernel,
    out_type=jax.ShapeDtypeStruct((NC, NS, 1, N), jnp.float32),
    mesh=_mesh,
    compiler_params=pltpu.CompilerParams(needs_layout_passes=False),
    scratch_types=[
        pltpu.VMEM((EPT,), jnp.int32),
        pltpu.VMEM((1, N), jnp.float32),
    ],
)
def _deg_kernel(dst_hbm, out_hbm, idx_v, acc_v):
    cid = lax.axis_index("c")
    sid = lax.axis_index("s")

    zero16 = jnp.zeros((16,), jnp.float32)
    zero16i = jnp.zeros((16,), jnp.int32)
    one16 = jnp.full((16,), 1.0, jnp.float32)

    def zfill(i, _):
        acc_v[0, pl.ds(i * 16, 16)] = zero16
        return 0

    lax.fori_loop(0, N // 16, zfill, 0)

    wid = cid * NS + sid
    pltpu.sync_copy(dst_hbm.at[pl.ds(wid * EPT, EPT)], idx_v)

    def body(j, _):
        idx = idx_v[pl.ds(j * 16, 16)]
        plsc.addupdate_scatter(acc_v, [zero16i, idx], one16)
        return 0

    lax.fori_loop(0, EPT // 16, body, 0)

    pltpu.sync_copy(acc_v, out_hbm.at[cid, sid])


@functools.partial(
    pl.kernel,
    out_type=jax.ShapeDtypeStruct((NC, N, D), jnp.float32),
    mesh=_mesh,
    scratch_types=[
        pltpu.VMEM((EPW,), jnp.int32),
        pltpu.VMEM((EPW,), jnp.int32),
        *([pltpu.VMEM((CA, D), jnp.float32)] * NB),
        *([pltpu.SemaphoreType.DMA] * (2 * NB)),
        pltpu.VMEM_SHARED((N, D), jnp.float32),
    ],
)
def _agg_kernel(y_hbm, src_hbm, dst_hbm, out_hbm, idx_s, idx_d, *bufs):
    rows = list(bufs[:NB])
    gsem = list(bufs[NB:2 * NB])
    ssem = list(bufs[2 * NB:3 * NB])
    z_sh = bufs[3 * NB]
    cid = lax.axis_index("c")
    sid = lax.axis_index("s")

    wid = cid * NS + sid
    pltpu.sync_copy(src_hbm.at[wid], idx_s)
    pltpu.sync_copy(dst_hbm.at[wid], idx_d)

    _tile_sliced_copy(
        sid,
        lambda off, n: y_hbm.at[pl.ds(off, n)],
        lambda off, n: z_sh.at[pl.ds(off, n)],
    )
    plsc.subcore_barrier()

    def g_start(r, q):
        pltpu.async_copy(
            y_hbm.at[idx_s.at[pl.ds(r * CA, CA)]], rows[q], gsem[q]
        )

    def s_start(r, q):
        pltpu.async_copy(
            rows[q], z_sh.at[idx_d.at[pl.ds(r * CA, CA)]], ssem[q], add=True
        )

    def g_wait(r, q):
        pltpu.make_async_copy(
            y_hbm.at[idx_s.at[pl.ds(r * CA, CA)]], rows[q], gsem[q]
        ).wait()

    def s_wait(r, q):
        pltpu.make_async_copy(
            rows[q], z_sh.at[idx_d.at[pl.ds(r * CA, CA)]], ssem[q]
        ).wait()

    for q in range(LK):
        g_start(q, q)

    def hex_body(j, _):
        r0 = NB * j
        for i in range(NB):
            ri = r0 + i
            qn = (i + LK) % NB
            g_wait(ri, i)
            s_start(ri, i)
            if i + LK >= NB:
                s_wait(ri + LK - NB, qn)
                g_start(ri + LK, qn)
            else:

                @pl.when(j > 0)
                def _(ri=ri, qn=qn):
                    s_wait(ri + LK - NB, qn)

                g_start(ri + LK, qn)
        return 0

    lax.fori_loop(0, KJ, hex_body, 0)

    for t in range(LK):
        g_wait(RA - LK + t, t)
        s_start(RA - LK + t, t)
    s_wait(RA - LK - 2, NB - 2)
    s_wait(RA - LK - 1, NB - 1)
    for t in range(LK):
        s_wait(RA - LK + t, t)
    plsc.subcore_barrier()

    _tile_sliced_copy(
        sid,
        lambda off, n: z_sh.at[pl.ds(off, n)],
        lambda off, n: out_hbm.at[cid, pl.ds(off, n)],
    )


_BR = 1000
_GRID = N // _BR


def _dinv_body(degp_ref, dinv_ref):
    deg = jnp.sum(degp_ref[...], axis=(0, 1, 2)) + 1.0
    dinv_ref[...] = lax.rsqrt(deg)[:, None]


def _lin1_body(x_ref, w_ref, dinv_ref, y_ref):
    y = jnp.dot(x_ref[...], w_ref[...], preferred_element_type=jnp.float32)
    y_ref[...] = y * dinv_ref[...]


def _lin2_body(z_ref, y_ref, dinv_ref, b_ref, w_ref, y2_ref):
    dinv = dinv_ref[...]
    z = z_ref[0] + z_ref[1] - y_ref[...]
    h = jnp.tanh(z * dinv + b_ref[...])
    y = jnp.dot(h, w_ref[...], preferred_element_type=jnp.float32)
    y2_ref[...] = y * dinv


def _head_body(z_ref, y_ref, dinv_ref, b_ref, wlt_ref, bl_ref, out_ref, lsm_ref):
    z = z_ref[0] + z_ref[1] - y_ref[...]
    h = jnp.tanh(z * dinv_ref[...] + b_ref[...])
    o = jnp.dot(h, wlt_ref[...], preferred_element_type=jnp.float32) + bl_ref[...]
    m = jnp.max(o, axis=1, keepdims=True)
    lse = jnp.log(jnp.sum(jnp.exp(o - m), axis=1, keepdims=True)) + m
    out_ref[...] = o
    lsm_ref[...] = o - lse


def _row_spec(width):
    return pl.BlockSpec((_BR, width), lambda i: (i, 0))


def _full_spec(shape):
    nz = len(shape)
    return pl.BlockSpec(shape, lambda i, _n=nz: (0,) * _n)


def _z_spec():
    return pl.BlockSpec((NC, _BR, D), lambda i: (0, i, 0))


def kernel(x, edge_index, W1, b1, W2, b2, Wl, bl):
    src = edge_index[0]
    dst = edge_index[1]
    src2 = src.reshape(NW, EPW)
    dst2 = dst.reshape(NW, EPW)
    b1r = b1.reshape(1, D)
    b2r = b2.reshape(1, D)
    blr = bl.reshape(1, D)
    wlt = Wl.T

    degp = _deg_kernel(dst)

    dinv = pl.pallas_call(
        _dinv_body,
        out_shape=jax.ShapeDtypeStruct((N, 1), jnp.float32),
    )(degp)

    y1 = pl.pallas_call(
        _lin1_body,
        grid=(_GRID,),
        in_specs=[_row_spec(D), _full_spec((D, D)), _row_spec(1)],
        out_specs=_row_spec(D),
        out_shape=jax.ShapeDtypeStruct((N, D), jnp.float32),
    )(x, W1, dinv)

    z1 = _agg_kernel(y1, src2, dst2)

    y2 = pl.pallas_call(
        _lin2_body,
        grid=(_GRID,),
        in_specs=[
            _z_spec(),
            _row_spec(D),
            _row_spec(1),
            _full_spec((1, D)),
            _full_spec((D, D)),
        ],
        out_specs=_row_spec(D),
        out_shape=jax.ShapeDtypeStruct((N, D), jnp.float32),
    )(z1, y1, dinv, b1r, W2)

    z2 = _agg_kernel(y2, src2, dst2)

    out, lsm = pl.pallas_call(
        _head_body,
        grid=(_GRID,),
        in_specs=[
            _z_spec(),
            _row_spec(D),
            _row_spec(1),
            _full_spec((1, D)),
            _full_spec((D, D)),
            _full_spec((1, D)),
        ],
        out_specs=[_row_spec(D), _row_spec(D)],
        out_shape=[
            jax.ShapeDtypeStruct((N, D), jnp.float32),
            jax.ShapeDtypeStruct((N, D), jnp.float32),
        ],
    )(z2, y2, dinv, b2r, wlt, blr)

    return (out, lsm)
